# Initial kernel scaffold; baseline (speedup 1.0000x reference)
import jax
import jax.numpy as jnp
from jax.experimental import pallas as pl


def kernel(x, edge_index, batch, W1, b1, gamma1, beta1, W2, b2, gamma2, beta2, Wlin, blin):
    raise NotImplementedError("write your pallas kernel here")


# speedup vs baseline: 13.8664x; 13.8664x over previous
import functools

import jax
import jax.numpy as jnp
from jax import lax
from jax.experimental import pallas as pl
from jax.experimental.pallas import tpu as pltpu
from jax.experimental.pallas import tpu_sc as plsc

N = 10000
E = 320000
D = 128
H = 128
C = 16
G = 128
EPS = 1e-5

NC = 2
NS = 16
NW = NC * NS
CH = 128
KCH = 79
EPAD = NW * KCH * CH
ROWS_PT = 632
ACC_ROWS = NS * ROWS_PT
DUMMY = N

_mesh = plsc.VectorSubcoreMesh(core_axis_name="c", subcore_axis_name="s")


@functools.partial(
    pl.kernel,
    out_type=jax.ShapeDtypeStruct((NC, ACC_ROWS, 16), jnp.float32),
    mesh=_mesh,
    scratch_types=[
        pltpu.VMEM((KCH, CH), jnp.int32),
        pltpu.VMEM_SHARED((ACC_ROWS, 16), jnp.float32),
    ],
)
def _sc_degree(dst_hbm, ones_hbm, zeros_hbm, out_hbm, didx, acc):
    c = lax.axis_index("c")
    s = lax.axis_index("s")
    wid = c * NS + s
    pltpu.sync_copy(zeros_hbm, acc.at[pl.ds(s * ROWS_PT, ROWS_PT)])
    pltpu.sync_copy(dst_hbm.at[wid], didx)

    def run(ones_v):
        pltpu.sync_copy(ones_hbm, ones_v)
        plsc.subcore_barrier()

        def body(k, carry):
            pltpu.sync_copy(ones_v, acc.at[didx.at[k]], add=True)
            return carry

        lax.fori_loop(0, KCH, body, 0)
        plsc.subcore_barrier()
        pltpu.sync_copy(acc.at[pl.ds(s * ROWS_PT, ROWS_PT)],
                        out_hbm.at[c, pl.ds(s * ROWS_PT, ROWS_PT)])

    pl.run_scoped(run, pltpu.VMEM((CH, 16), jnp.float32))


@functools.partial(
    pl.kernel,
    out_type=jax.ShapeDtypeStruct((NC, ACC_ROWS, H), jnp.float32),
    mesh=_mesh,
    scratch_types=[
        pltpu.VMEM((KCH, CH), jnp.int32),
        pltpu.VMEM((KCH, CH), jnp.int32),
        pltpu.VMEM((CH, H), jnp.float32),
        pltpu.VMEM_SHARED((ACC_ROWS, H), jnp.float32),
        pltpu.SemaphoreType.DMA,
    ],
)
def _sc_agg(xws_hbm, src_hbm, dst_hbm, zeros_hbm, out_hbm,
            sidx, didx, rows, acc, sem):
    c = lax.axis_index("c")
    s = lax.axis_index("s")
    wid = c * NS + s
    pltpu.sync_copy(zeros_hbm, acc.at[pl.ds(s * ROWS_PT, ROWS_PT)])
    pltpu.sync_copy(src_hbm.at[wid], sidx)
    pltpu.sync_copy(dst_hbm.at[wid], didx)
    plsc.subcore_barrier()

    def body(k, carry):
        pltpu.async_copy(xws_hbm.at[sidx.at[k]], rows, sem).wait()
        pltpu.sync_copy(rows, acc.at[didx.at[k]], add=True)
        return carry

    lax.fori_loop(0, KCH, body, 0)
    plsc.subcore_barrier()
    pltpu.sync_copy(acc.at[pl.ds(s * ROWS_PT, ROWS_PT)],
                    out_hbm.at[c, pl.ds(s * ROWS_PT, ROWS_PT)])


def _tc_prep_body(x_ref, w1_ref, cnt_ref, xws_ref, dinv_ref):
    cnt = cnt_ref[0] + cnt_ref[1]
    deg = cnt[:N, 0:1] + 1.0
    dinv = lax.rsqrt(deg)
    xw = jnp.dot(x_ref[...], w1_ref[...], preferred_element_type=jnp.float32)
    xws_ref[...] = xw * dinv
    dinv_ref[...] = dinv


def _bn_relu(h, g, be):
    mean = jnp.mean(h, axis=0, keepdims=True)
    var = jnp.mean((h - mean) ** 2, axis=0, keepdims=True)
    return jnp.maximum((h - mean) * lax.rsqrt(var + EPS) * g + be, 0.0)


def _tc_mid_body(agg_ref, xws_ref, dinv_ref, b_ref, g_ref, be_ref, w2_ref,
                 out_ref):
    agg = agg_ref[0, :N, :] + agg_ref[1, :N, :]
    dinv = dinv_ref[...]
    h = dinv * (agg + xws_ref[...]) + b_ref[...]
    hn = _bn_relu(h, g_ref[...], be_ref[...])
    out_ref[...] = jnp.dot(hn, w2_ref[...],
                           preferred_element_type=jnp.float32) * dinv


def _tc_fin_body(agg_ref, xws_ref, dinv_ref, b_ref, g_ref, be_ref,
                 batch_ref, wl_ref, bl_ref, out_ref):
    agg = agg_ref[0, :N, :] + agg_ref[1, :N, :]
    dinv = dinv_ref[...]
    h = dinv * (agg + xws_ref[...]) + b_ref[...]
    hn = _bn_relu(h, g_ref[...], be_ref[...])
    oh = (batch_ref[...] ==
          lax.broadcasted_iota(jnp.int32, (N, G), 1)).astype(jnp.float32)
    psum = lax.dot_general(oh, hn, (((0,), (0,)), ((), ())),
                           preferred_element_type=jnp.float32)
    cg = jnp.sum(oh, axis=0)[:, None]
    pooled = psum / jnp.maximum(cg, 1.0)
    out_ref[...] = jnp.dot(pooled, wl_ref[...],
                           preferred_element_type=jnp.float32) + bl_ref[...]


_tc_prep = pl.pallas_call(
    _tc_prep_body,
    out_shape=[jax.ShapeDtypeStruct((N, H), jnp.float32),
               jax.ShapeDtypeStruct((N, 1), jnp.float32)],
)

_tc_mid = pl.pallas_call(
    _tc_mid_body,
    out_shape=jax.ShapeDtypeStruct((N, H), jnp.float32),
)

_tc_fin = pl.pallas_call(
    _tc_fin_body,
    out_shape=jax.ShapeDtypeStruct((G, C), jnp.float32),
)


def kernel(x, edge_index, batch, W1, b1, gamma1, beta1, W2, b2, gamma2,
           beta2, Wlin, blin):
    src = edge_index[0]
    dst = edge_index[1]
    pad = EPAD - E
    src_r = jnp.concatenate(
        [src, jnp.zeros((pad,), jnp.int32)]).reshape(NW, KCH, CH)
    dst_r = jnp.concatenate(
        [dst, jnp.full((pad,), DUMMY, jnp.int32)]).reshape(NW, KCH, CH)
    ones16 = jnp.ones((CH, 16), jnp.float32)
    z16 = jnp.zeros((ROWS_PT, 16), jnp.float32)
    z128 = jnp.zeros((ROWS_PT, H), jnp.float32)

    cnt = _sc_degree(dst_r, ones16, z16)
    xws1, dinv = _tc_prep(x, W1, cnt)
    agg1 = _sc_agg(xws1, src_r, dst_r, z128)
    xws2 = _tc_mid(agg1, xws1, dinv, b1.reshape(1, H), gamma1.reshape(1, H),
                   beta1.reshape(1, H), W2)
    agg2 = _sc_agg(xws2, src_r, dst_r, z128)
    out = _tc_fin(agg2, xws2, dinv, b2.reshape(1, H), gamma2.reshape(1, H),
                  beta2.reshape(1, H), batch.reshape(N, 1), Wlin,
                  blin.reshape(1, C))
    return out
